# Initial kernel scaffold; baseline (speedup 1.0000x reference)
import functools

import jax
import jax.numpy as jnp
from jax import lax
from jax.experimental import pallas as pl
from jax.experimental.pallas import tpu as pltpu
from jax.experimental.pallas import tpu_sc as plsc

N = 10000
E = 640000
G = 64

NC = 2
NS = 16
NW = NC * NS
CHUNK = 128
CPW = 160
EP = NW * CPW * CHUNK
PAD_SPREAD = 240
N_PAD = 10240
RPT = N_PAD // NS

_mesh = plsc.VectorSubcoreMesh(core_axis_name="c", subcore_axis_name="s")


def _zero_vmem(ref, rows, width):
  zero = jnp.zeros((16,), jnp.float32)

  def body(i, carry):
    for j in range(width // 16):
      ref[i, pl.ds(j * 16, 16)] = zero
    return carry

  lax.fori_loop(0, rows, body, 0)


def _make_sc_agg(d):
  @functools.partial(
      pl.kernel,
      out_type=jax.ShapeDtypeStruct((NC, N_PAD, d), jnp.float32),
      mesh=_mesh,
      scratch_types=[
          pltpu.VMEM((CPW, CHUNK), jnp.int32),
          pltpu.VMEM((CPW, CHUNK), jnp.int32),
          pltpu.VMEM((CHUNK, d), jnp.float32),
          pltpu.VMEM_SHARED((N_PAD, d), jnp.float32),
          pltpu.SemaphoreType.DMA,
      ],
  )
  def agg(z_hbm, rows_hbm, cols_hbm, out_hbm, idxr_v, idxc_v, rows_v, acc,
          sem):
    c = lax.axis_index("c")
    s = lax.axis_index("s")
    w = s * NC + c

    _zero_vmem(rows_v, CHUNK, d)

    def zbody(i, carry):
      pltpu.sync_copy(rows_v, acc.at[pl.ds(s * RPT + i * CHUNK, CHUNK)])
      return carry

    lax.fori_loop(0, RPT // CHUNK, zbody, 0)

    pltpu.sync_copy(rows_hbm.at[w], idxr_v)
    pltpu.sync_copy(cols_hbm.at[w], idxc_v)
    plsc.subcore_barrier()

    def ebody(j, carry):
      pltpu.async_copy(z_hbm.at[idxr_v.at[j]], rows_v, sem).wait()
      pltpu.sync_copy(rows_v, acc.at[idxc_v.at[j]], add=True)
      return carry

    lax.fori_loop(0, CPW, ebody, 0)
    plsc.subcore_barrier()

    def wbody(i, carry):
      off = s * RPT + i * CHUNK
      pltpu.sync_copy(acc.at[pl.ds(off, CHUNK)], rows_v)
      pltpu.sync_copy(rows_v, out_hbm.at[c, pl.ds(off, CHUNK)])
      return carry

    lax.fori_loop(0, RPT // CHUNK, wbody, 0)

  return agg


_sc_agg = {d: _make_sc_agg(d) for d in (16, 32, 64)}


@functools.partial(
    pl.kernel,
    out_type=jax.ShapeDtypeStruct((NC, N_PAD), jnp.float32),
    mesh=_mesh,
    scratch_types=[
        pltpu.VMEM((CPW, CHUNK), jnp.int32),
        pltpu.VMEM((CHUNK,), jnp.float32),
        pltpu.VMEM((CHUNK,), jnp.float32),
        pltpu.VMEM_SHARED((N_PAD,), jnp.float32),
    ],
)
def _sc_deg(cols_hbm, out_hbm, idxc_v, ones_v, buf_v, acc):
  c = lax.axis_index("c")
  s = lax.axis_index("s")
  w = s * NC + c

  one = jnp.ones((16,), jnp.float32)
  zero = jnp.zeros((16,), jnp.float32)
  for j in range(CHUNK // 16):
    ones_v[pl.ds(j * 16, 16)] = one
    buf_v[pl.ds(j * 16, 16)] = zero

  def zbody(i, carry):
    pltpu.sync_copy(buf_v, acc.at[pl.ds(s * RPT + i * CHUNK, CHUNK)])
    return carry

  lax.fori_loop(0, RPT // CHUNK, zbody, 0)
  pltpu.sync_copy(cols_hbm.at[w], idxc_v)
  plsc.subcore_barrier()

  def ebody(j, carry):
    pltpu.sync_copy(ones_v, acc.at[idxc_v.at[j]], add=True)
    return carry

  lax.fori_loop(0, CPW, ebody, 0)
  plsc.subcore_barrier()

  def wbody(i, carry):
    off = s * RPT + i * CHUNK
    pltpu.sync_copy(acc.at[pl.ds(off, CHUNK)], buf_v)
    pltpu.sync_copy(buf_v, out_hbm.at[c, pl.ds(off, CHUNK)])
    return carry

  lax.fori_loop(0, RPT // CHUNK, wbody, 0)


def _tc_call(fn, out_shapes):
  return pl.pallas_call(fn, out_shape=out_shapes)


def _tc1_body(deg_ref, x_ref, dinv_ref, xp_ref):
  deg = deg_ref[:, 0:1] + deg_ref[:, 1:2] + 1.0
  rows = lax.broadcasted_iota(jnp.int32, (N_PAD, 1), 0)
  dinv = jnp.where(rows < N, lax.rsqrt(deg), 0.0)
  dinv_ref[...] = dinv
  xp_ref[...] = dinv * x_ref[...]


def _tc2_body(p_ref, xp_ref, dinv_ref, w1_ref, b1_ref, w2_ref, out_ref):
  dinv = dinv_ref[...]
  agg = dinv * (p_ref[0] + p_ref[1] + xp_ref[...])
  h = jnp.maximum(
      jnp.dot(agg, w1_ref[...], preferred_element_type=jnp.float32)
      + b1_ref[...], 0.0)
  out_ref[...] = dinv * jnp.dot(
      h, w2_ref[...], preferred_element_type=jnp.float32)


def _tc_mid_body(p_ref, z_ref, dinv_ref, b_ref, w_ref, out_ref):
  dinv = dinv_ref[...]
  agg = dinv * (p_ref[0] + p_ref[1] + z_ref[...])
  h = jnp.maximum(agg + b_ref[...], 0.0)
  out_ref[...] = dinv * jnp.dot(
      h, w_ref[...], preferred_element_type=jnp.float32)


def _tc5_body(p_ref, z_ref, dinv_ref, b_ref, batch_ref, out_ref):
  h4 = dinv_ref[...] * (p_ref[0] + p_ref[1] + z_ref[...]) + b_ref[...]
  cols = lax.broadcasted_iota(jnp.int32, (N_PAD, 16), 1)
  h4 = jnp.where(cols == 15, 1.0, h4)
  graphs = lax.broadcasted_iota(jnp.int32, (N_PAD, G), 1)
  p = (batch_ref[...] == graphs).astype(jnp.float32)
  sums = lax.dot_general(p, h4, (((0,), (0,)), ((), ())),
                         preferred_element_type=jnp.float32)
  counts = sums[:, 15:16]
  mean = sums[:, 0:3] / jnp.maximum(counts, 1.0)
  zc = mean - jnp.max(mean, axis=1, keepdims=True)
  out_ref[...] = zc - jnp.log(jnp.sum(jnp.exp(zc), axis=1, keepdims=True))


def kernel(x, edge_index, batch, W1, b1, W2, b2, W3, b3, W4, b4):
  f32 = jnp.float32
  row = edge_index[0].astype(jnp.int32)
  col = edge_index[1].astype(jnp.int32)
  pad_idx = N + (jnp.arange(EP - E, dtype=jnp.int32) % PAD_SPREAD)
  rows_p = jnp.concatenate([row, pad_idx]).reshape(NW, CPW, CHUNK)
  cols_p = jnp.concatenate([col, pad_idx]).reshape(NW, CPW, CHUNK)

  x_pad = jnp.pad(x.astype(f32), ((0, N_PAD - N), (0, 16 - x.shape[1])))
  batch_pad = jnp.pad(batch.astype(jnp.int32), (0, N_PAD - N),
                      constant_values=G).reshape(N_PAD, 1)
  w1p = jnp.pad(W1, ((0, 16 - W1.shape[0]), (0, 0)))
  b1r = b1.reshape(1, 128)
  b2r = b2.reshape(1, 64)
  w3p = jnp.pad(W3, ((0, 0), (0, 32 - W3.shape[1])))
  b3p = jnp.pad(b3, (0, 32 - b3.shape[0])).reshape(1, 32)
  w4p = jnp.pad(W4, ((0, 32 - W4.shape[0]), (0, 16 - W4.shape[1])))
  b4p = jnp.pad(b4, (0, 16 - b4.shape[0])).reshape(1, 16)

  deg_p = _sc_deg(cols_p)
  deg_t = deg_p.T

  dinv, xp = _tc_call(
      _tc1_body,
      (jax.ShapeDtypeStruct((N_PAD, 1), f32),
       jax.ShapeDtypeStruct((N_PAD, 16), f32)))(deg_t, x_pad)

  p1 = _sc_agg[16](xp, rows_p, cols_p)
  z2 = _tc_call(
      _tc2_body,
      jax.ShapeDtypeStruct((N_PAD, 64), f32))(p1, xp, dinv, w1p, b1r, W2)

  p2 = _sc_agg[64](z2, rows_p, cols_p)
  z3 = _tc_call(
      _tc_mid_body,
      jax.ShapeDtypeStruct((N_PAD, 32), f32))(p2, z2, dinv, b2r, w3p)

  p3 = _sc_agg[32](z3, rows_p, cols_p)
  z4 = _tc_call(
      _tc_mid_body,
      jax.ShapeDtypeStruct((N_PAD, 16), f32))(p3, z3, dinv, b3p, w4p)

  p4 = _sc_agg[16](z4, rows_p, cols_p)
  out = _tc_call(
      _tc5_body,
      jax.ShapeDtypeStruct((G, 3), f32))(p4, z4, dinv, b4p, batch_pad)
  return out


# speedup vs baseline: 45.6348x; 45.6348x over previous
import functools

import jax
import jax.numpy as jnp
from jax import lax
from jax.experimental import pallas as pl
from jax.experimental.pallas import tpu as pltpu
from jax.experimental.pallas import tpu_sc as plsc

N = 10000
E = 640000
G = 64

NC = 2
NS = 16
NW = NC * NS
CHUNK = 128
CPW = 160
EP = NW * CPW * CHUNK
PAD_SPREAD = 240
N_PAD = 10240
RPT = N_PAD // NS

_mesh = plsc.VectorSubcoreMesh(core_axis_name="c", subcore_axis_name="s")


def _zero_vmem(ref, rows, width):
  zero = jnp.zeros((16,), jnp.float32)

  def body(i, carry):
    for j in range(width // 16):
      ref[i, pl.ds(j * 16, 16)] = zero
    return carry

  lax.fori_loop(0, rows, body, 0)


def _make_sc_agg(d, fp):
  @functools.partial(
      pl.kernel,
      out_type=jax.ShapeDtypeStruct((NC, fp, N_PAD, d), jnp.float32),
      mesh=_mesh,
      compiler_params=pltpu.CompilerParams(use_tc_tiling_on_sc=False),
      scratch_types=[
          pltpu.VMEM((CPW, CHUNK), jnp.int32),
          pltpu.VMEM((CPW, CHUNK), jnp.int32),
          pltpu.VMEM((CHUNK, d), jnp.float32),
          pltpu.VMEM_SHARED((N_PAD, d), jnp.float32),
          pltpu.VMEM_SHARED((N_PAD, d), jnp.float32),
          pltpu.SemaphoreType.DMA,
      ],
  )
  def agg(z_hbm, rows_hbm, cols_hbm, out_hbm, idxr_v, idxc_v, rows_v, z_sp,
          acc, sem):
    c = lax.axis_index("c")
    s = lax.axis_index("s")
    w = s * NC + c

    pltpu.sync_copy(rows_hbm.at[w], idxr_v)
    pltpu.sync_copy(cols_hbm.at[w], idxc_v)

    for f in range(fp):
      pltpu.sync_copy(z_hbm.at[f, pl.ds(s * RPT, RPT)],
                      z_sp.at[pl.ds(s * RPT, RPT)])
      _zero_vmem(rows_v, CHUNK, d)

      def zbody(i, carry):
        pltpu.sync_copy(rows_v, acc.at[pl.ds(s * RPT + i * CHUNK, CHUNK)])
        return carry

      lax.fori_loop(0, RPT // CHUNK, zbody, 0)
      plsc.subcore_barrier()

      def ebody(j, carry):
        pltpu.async_copy(z_sp.at[idxr_v.at[j]], rows_v, sem).wait()
        pltpu.sync_copy(rows_v, acc.at[idxc_v.at[j]], add=True)
        return carry

      lax.fori_loop(0, CPW, ebody, 0)
      plsc.subcore_barrier()

      def wbody(i, carry):
        off = s * RPT + i * CHUNK
        pltpu.sync_copy(acc.at[pl.ds(off, CHUNK)], rows_v)
        pltpu.sync_copy(rows_v, out_hbm.at[c, f, pl.ds(off, CHUNK)])
        return carry

      lax.fori_loop(0, RPT // CHUNK, wbody, 0)
      plsc.subcore_barrier()

  return agg


_agg16 = _make_sc_agg(16, 1)
_agg32 = _make_sc_agg(32, 1)
_agg32x2 = _make_sc_agg(32, 2)


@functools.partial(
    pl.kernel,
    out_type=jax.ShapeDtypeStruct((NC, N_PAD), jnp.float32),
    mesh=_mesh,
    compiler_params=pltpu.CompilerParams(use_tc_tiling_on_sc=False),
    scratch_types=[
        pltpu.VMEM((CPW, CHUNK), jnp.int32),
        pltpu.VMEM((CHUNK,), jnp.float32),
        pltpu.VMEM((CHUNK,), jnp.float32),
        pltpu.VMEM_SHARED((N_PAD,), jnp.float32),
    ],
)
def _sc_deg(cols_hbm, out_hbm, idxc_v, ones_v, buf_v, acc):
  c = lax.axis_index("c")
  s = lax.axis_index("s")
  w = s * NC + c

  one = jnp.ones((16,), jnp.float32)
  zero = jnp.zeros((16,), jnp.float32)
  for j in range(CHUNK // 16):
    ones_v[pl.ds(j * 16, 16)] = one
    buf_v[pl.ds(j * 16, 16)] = zero

  def zbody(i, carry):
    pltpu.sync_copy(buf_v, acc.at[pl.ds(s * RPT + i * CHUNK, CHUNK)])
    return carry

  lax.fori_loop(0, RPT // CHUNK, zbody, 0)
  pltpu.sync_copy(cols_hbm.at[w], idxc_v)
  plsc.subcore_barrier()

  def ebody(j, carry):
    pltpu.sync_copy(ones_v, acc.at[idxc_v.at[j]], add=True)
    return carry

  lax.fori_loop(0, CPW, ebody, 0)
  plsc.subcore_barrier()

  def wbody(i, carry):
    off = s * RPT + i * CHUNK
    pltpu.sync_copy(acc.at[pl.ds(off, CHUNK)], buf_v)
    pltpu.sync_copy(buf_v, out_hbm.at[c, pl.ds(off, CHUNK)])
    return carry

  lax.fori_loop(0, RPT // CHUNK, wbody, 0)


def _tc_call(fn, out_shapes):
  return pl.pallas_call(fn, out_shape=out_shapes)


def _tc1_body(deg_ref, x_ref, dinv_ref, xp_ref):
  deg = deg_ref[:, 0:1] + deg_ref[:, 1:2] + 1.0
  rows = lax.broadcasted_iota(jnp.int32, (N_PAD, 1), 0)
  dinv = jnp.where(rows < N, lax.rsqrt(deg), 0.0)
  dinv_ref[...] = dinv
  xp_ref[0] = dinv * x_ref[...]


def _tc2_body(p_ref, xp_ref, dinv_ref, w1_ref, b1_ref, w2_ref, out_ref):
  dinv = dinv_ref[...]
  agg = dinv * (p_ref[0, 0] + p_ref[1, 0] + xp_ref[0])
  h = jnp.maximum(
      jnp.dot(agg, w1_ref[...], preferred_element_type=jnp.float32)
      + b1_ref[...], 0.0)
  out_ref[0] = dinv * jnp.dot(
      h, w2_ref[0], preferred_element_type=jnp.float32)
  out_ref[1] = dinv * jnp.dot(
      h, w2_ref[1], preferred_element_type=jnp.float32)


def _tc3_body(p_ref, z_ref, dinv_ref, b_ref, w_ref, out_ref):
  dinv = dinv_ref[...]
  agg = jnp.concatenate(
      [dinv * (p_ref[0, 0] + p_ref[1, 0] + z_ref[0]),
       dinv * (p_ref[0, 1] + p_ref[1, 1] + z_ref[1])], axis=1)
  h = jnp.maximum(agg + b_ref[...], 0.0)
  out_ref[0] = dinv * jnp.dot(
      h, w_ref[...], preferred_element_type=jnp.float32)


def _tc4_body(p_ref, z_ref, dinv_ref, b_ref, w_ref, out_ref):
  dinv = dinv_ref[...]
  agg = dinv * (p_ref[0, 0] + p_ref[1, 0] + z_ref[0])
  h = jnp.maximum(agg + b_ref[...], 0.0)
  out_ref[0] = dinv * jnp.dot(
      h, w_ref[...], preferred_element_type=jnp.float32)


def _tc5_body(p_ref, z_ref, dinv_ref, b_ref, batch_ref, out_ref):
  h4 = dinv_ref[...] * (p_ref[0, 0] + p_ref[1, 0] + z_ref[0]) + b_ref[...]
  cols = lax.broadcasted_iota(jnp.int32, (N_PAD, 16), 1)
  h4 = jnp.where(cols == 15, 1.0, h4)
  graphs = lax.broadcasted_iota(jnp.int32, (N_PAD, G), 1)
  p = (batch_ref[...] == graphs).astype(jnp.float32)
  sums = lax.dot_general(p, h4, (((0,), (0,)), ((), ())),
                         preferred_element_type=jnp.float32)
  counts = sums[:, 15:16]
  mean = sums[:, 0:3] / jnp.maximum(counts, 1.0)
  zc = mean - jnp.max(mean, axis=1, keepdims=True)
  out_ref[...] = zc - jnp.log(jnp.sum(jnp.exp(zc), axis=1, keepdims=True))


def kernel(x, edge_index, batch, W1, b1, W2, b2, W3, b3, W4, b4):
  f32 = jnp.float32
  row = edge_index[0].astype(jnp.int32)
  col = edge_index[1].astype(jnp.int32)
  pad_idx = N + (jnp.arange(EP - E, dtype=jnp.int32) % PAD_SPREAD)
  rows_p = jnp.concatenate([row, pad_idx]).reshape(NW, CPW, CHUNK)
  cols_p = jnp.concatenate([col, pad_idx]).reshape(NW, CPW, CHUNK)

  x_pad = jnp.pad(x.astype(f32), ((0, N_PAD - N), (0, 16 - x.shape[1])))
  batch_pad = jnp.pad(batch.astype(jnp.int32), (0, N_PAD - N),
                      constant_values=G).reshape(N_PAD, 1)
  w1p = jnp.pad(W1, ((0, 16 - W1.shape[0]), (0, 0)))
  b1r = b1.reshape(1, 128)
  b2r = b2.reshape(1, 64)
  w2s = jnp.stack([W2[:, :32], W2[:, 32:]])
  w3p = jnp.pad(W3, ((0, 0), (0, 32 - W3.shape[1])))
  b3p = jnp.pad(b3, (0, 32 - b3.shape[0])).reshape(1, 32)
  w4p = jnp.pad(W4, ((0, 32 - W4.shape[0]), (0, 16 - W4.shape[1])))
  b4p = jnp.pad(b4, (0, 16 - b4.shape[0])).reshape(1, 16)

  deg_p = _sc_deg(cols_p)
  deg_t = deg_p.T

  dinv, xp = _tc_call(
      _tc1_body,
      (jax.ShapeDtypeStruct((N_PAD, 1), f32),
       jax.ShapeDtypeStruct((1, N_PAD, 16), f32)))(deg_t, x_pad)

  p1 = _agg16(xp, rows_p, cols_p)
  z2 = _tc_call(
      _tc2_body,
      jax.ShapeDtypeStruct((2, N_PAD, 32), f32))(p1, xp, dinv, w1p, b1r, w2s)

  p2 = _agg32x2(z2, rows_p, cols_p)
  z3 = _tc_call(
      _tc3_body,
      jax.ShapeDtypeStruct((1, N_PAD, 32), f32))(p2, z2, dinv, b2r, w3p)

  p3 = _agg32(z3, rows_p, cols_p)
  z4 = _tc_call(
      _tc4_body,
      jax.ShapeDtypeStruct((1, N_PAD, 16), f32))(p3, z3, dinv, b3p, w4p)

  p4 = _agg16(z4, rows_p, cols_p)
  out = _tc_call(
      _tc5_body,
      jax.ShapeDtypeStruct((G, 3), f32))(p4, z4, dinv, b4p, batch_pad)
  return out


# speedup vs baseline: 54.3034x; 1.1900x over previous
import functools

import jax
import jax.numpy as jnp
from jax import lax
from jax.experimental import pallas as pl
from jax.experimental.pallas import tpu as pltpu
from jax.experimental.pallas import tpu_sc as plsc

N = 10000
E = 640000
G = 64

NC = 2
NS = 16
NW = NC * NS
CHUNK = 512
CPW = 40
EP = NW * CPW * CHUNK
PAD_SPREAD = 240
N_PAD = 10240
RPT = N_PAD // NS

_mesh = plsc.VectorSubcoreMesh(core_axis_name="c", subcore_axis_name="s")


def _zero_vmem(ref, rows, width):
  zero = jnp.zeros((16,), jnp.float32)

  def body(i, carry):
    for j in range(width // 16):
      ref[i, pl.ds(j * 16, 16)] = zero
    return carry

  lax.fori_loop(0, rows, body, 0)


def _make_sc_agg(d, fp):
  @functools.partial(
      pl.kernel,
      out_type=jax.ShapeDtypeStruct((NC, fp, N_PAD, d), jnp.float32),
      mesh=_mesh,
      compiler_params=pltpu.CompilerParams(use_tc_tiling_on_sc=False),
      scratch_types=[
          pltpu.VMEM((CPW, CHUNK), jnp.int32),
          pltpu.VMEM((CPW, CHUNK), jnp.int32),
          pltpu.VMEM((CHUNK, d), jnp.float32),
          pltpu.VMEM((CHUNK, d), jnp.float32),
          pltpu.VMEM_SHARED((N_PAD, d), jnp.float32),
          pltpu.VMEM_SHARED((N_PAD, d), jnp.float32),
          pltpu.SemaphoreType.DMA,
          pltpu.SemaphoreType.DMA,
          pltpu.SemaphoreType.DMA,
          pltpu.SemaphoreType.DMA,
      ],
  )
  def agg(z_hbm, rows_hbm, cols_hbm, out_hbm, idxr_v, idxc_v, buf_a, buf_b,
          z_sp, acc, gsa, gsb, ssa, ssb):
    c = lax.axis_index("c")
    s = lax.axis_index("s")
    w = s * NC + c

    pltpu.sync_copy(rows_hbm.at[w], idxr_v)
    pltpu.sync_copy(cols_hbm.at[w], idxc_v)

    def gstart(j, buf, sem):
      pltpu.async_copy(z_sp.at[idxr_v.at[j]], buf, sem)

    def gwait(j, buf, sem):
      pltpu.make_async_copy(z_sp.at[idxr_v.at[j]], buf, sem).wait()

    def sstart(j, buf, sem):
      pltpu.async_copy(buf, acc.at[idxc_v.at[j]], sem, add=True)

    def swait(j, buf, sem):
      pltpu.make_async_copy(buf, acc.at[idxc_v.at[j]], sem).wait()

    for f in range(fp):
      pltpu.sync_copy(z_hbm.at[f, pl.ds(s * RPT, RPT // 2)],
                      z_sp.at[pl.ds(s * RPT, RPT // 2)])
      pltpu.sync_copy(z_hbm.at[f, pl.ds(s * RPT + RPT // 2, RPT // 2)],
                      z_sp.at[pl.ds(s * RPT + RPT // 2, RPT // 2)])
      _zero_vmem(buf_a, CHUNK, d)
      pltpu.sync_copy(buf_a, acc.at[pl.ds(s * RPT, CHUNK)])
      pltpu.sync_copy(buf_a.at[pl.ds(0, RPT - CHUNK)],
                      acc.at[pl.ds(s * RPT + CHUNK, RPT - CHUNK)])
      plsc.subcore_barrier()

      gstart(0, buf_a, gsa)
      gstart(1, buf_b, gsb)
      gwait(0, buf_a, gsa)
      sstart(0, buf_a, ssa)
      gwait(1, buf_b, gsb)
      sstart(1, buf_b, ssb)

      def ebody(i, carry):
        j0 = 2 * i
        j1 = j0 + 1
        swait(j0 - 2, buf_a, ssa)
        gstart(j0, buf_a, gsa)
        swait(j1 - 2, buf_b, ssb)
        gstart(j1, buf_b, gsb)
        gwait(j0, buf_a, gsa)
        sstart(j0, buf_a, ssa)
        gwait(j1, buf_b, gsb)
        sstart(j1, buf_b, ssb)
        return carry

      lax.fori_loop(1, CPW // 2, ebody, 0)
      swait(CPW - 2, buf_a, ssa)
      swait(CPW - 1, buf_b, ssb)
      plsc.subcore_barrier()

      off = s * RPT
      pltpu.sync_copy(acc.at[pl.ds(off, CHUNK)], buf_a)
      pltpu.sync_copy(buf_a, out_hbm.at[c, f, pl.ds(off, CHUNK)])
      pltpu.sync_copy(acc.at[pl.ds(off + CHUNK, RPT - CHUNK)],
                      buf_b.at[pl.ds(0, RPT - CHUNK)])
      pltpu.sync_copy(buf_b.at[pl.ds(0, RPT - CHUNK)],
                      out_hbm.at[c, f, pl.ds(off + CHUNK, RPT - CHUNK)])
      plsc.subcore_barrier()

  return agg


_agg16 = _make_sc_agg(16, 1)
_agg32 = _make_sc_agg(32, 1)
_agg32x2 = _make_sc_agg(32, 2)


@functools.partial(
    pl.kernel,
    out_type=jax.ShapeDtypeStruct((NC, N_PAD), jnp.float32),
    mesh=_mesh,
    compiler_params=pltpu.CompilerParams(use_tc_tiling_on_sc=False),
    scratch_types=[
        pltpu.VMEM((CPW, CHUNK), jnp.int32),
        pltpu.VMEM((CHUNK,), jnp.float32),
        pltpu.VMEM((CHUNK,), jnp.float32),
        pltpu.VMEM_SHARED((N_PAD,), jnp.float32),
    ],
)
def _sc_deg(cols_hbm, out_hbm, idxc_v, ones_v, buf_v, acc):
  c = lax.axis_index("c")
  s = lax.axis_index("s")
  w = s * NC + c

  one = jnp.ones((16,), jnp.float32)
  zero = jnp.zeros((16,), jnp.float32)
  for j in range(CHUNK // 16):
    ones_v[pl.ds(j * 16, 16)] = one
    buf_v[pl.ds(j * 16, 16)] = zero

  pltpu.sync_copy(buf_v, acc.at[pl.ds(s * RPT, CHUNK)])
  pltpu.sync_copy(buf_v.at[pl.ds(0, RPT - CHUNK)],
                  acc.at[pl.ds(s * RPT + CHUNK, RPT - CHUNK)])
  pltpu.sync_copy(cols_hbm.at[w], idxc_v)
  plsc.subcore_barrier()

  def ebody(j, carry):
    pltpu.sync_copy(ones_v, acc.at[idxc_v.at[j]], add=True)
    return carry

  lax.fori_loop(0, CPW, ebody, 0)
  plsc.subcore_barrier()

  off = s * RPT
  pltpu.sync_copy(acc.at[pl.ds(off, CHUNK)], buf_v)
  pltpu.sync_copy(buf_v, out_hbm.at[c, pl.ds(off, CHUNK)])
  pltpu.sync_copy(acc.at[pl.ds(off + CHUNK, RPT - CHUNK)],
                  buf_v.at[pl.ds(0, RPT - CHUNK)])
  pltpu.sync_copy(buf_v.at[pl.ds(0, RPT - CHUNK)],
                  out_hbm.at[c, pl.ds(off + CHUNK, RPT - CHUNK)])


def _tc_call(fn, out_shapes):
  return pl.pallas_call(fn, out_shape=out_shapes)


def _tc1_body(deg_ref, x_ref, dinv_ref, xp_ref):
  deg = deg_ref[:, 0:1] + deg_ref[:, 1:2] + 1.0
  rows = lax.broadcasted_iota(jnp.int32, (N_PAD, 1), 0)
  dinv = jnp.where(rows < N, lax.rsqrt(deg), 0.0)
  dinv_ref[...] = dinv
  xp_ref[0] = dinv * x_ref[...]


def _tc2_body(p_ref, xp_ref, dinv_ref, w1_ref, b1_ref, w2_ref, out_ref):
  dinv = dinv_ref[...]
  agg = dinv * (p_ref[0, 0] + p_ref[1, 0] + xp_ref[0])
  h = jnp.maximum(
      jnp.dot(agg, w1_ref[...], preferred_element_type=jnp.float32)
      + b1_ref[...], 0.0)
  out_ref[0] = dinv * jnp.dot(
      h, w2_ref[0], preferred_element_type=jnp.float32)
  out_ref[1] = dinv * jnp.dot(
      h, w2_ref[1], preferred_element_type=jnp.float32)


def _tc3_body(p_ref, z_ref, dinv_ref, b_ref, w_ref, out_ref):
  dinv = dinv_ref[...]
  agg = jnp.concatenate(
      [dinv * (p_ref[0, 0] + p_ref[1, 0] + z_ref[0]),
       dinv * (p_ref[0, 1] + p_ref[1, 1] + z_ref[1])], axis=1)
  h = jnp.maximum(agg + b_ref[...], 0.0)
  out_ref[0] = dinv * jnp.dot(
      h, w_ref[...], preferred_element_type=jnp.float32)


def _tc4_body(p_ref, z_ref, dinv_ref, b_ref, w_ref, out_ref):
  dinv = dinv_ref[...]
  agg = dinv * (p_ref[0, 0] + p_ref[1, 0] + z_ref[0])
  h = jnp.maximum(agg + b_ref[...], 0.0)
  out_ref[0] = dinv * jnp.dot(
      h, w_ref[...], preferred_element_type=jnp.float32)


def _tc5_body(p_ref, z_ref, dinv_ref, b_ref, batch_ref, out_ref):
  h4 = dinv_ref[...] * (p_ref[0, 0] + p_ref[1, 0] + z_ref[0]) + b_ref[...]
  cols = lax.broadcasted_iota(jnp.int32, (N_PAD, 16), 1)
  h4 = jnp.where(cols == 15, 1.0, h4)
  graphs = lax.broadcasted_iota(jnp.int32, (N_PAD, G), 1)
  p = (batch_ref[...] == graphs).astype(jnp.float32)
  sums = lax.dot_general(p, h4, (((0,), (0,)), ((), ())),
                         preferred_element_type=jnp.float32)
  counts = sums[:, 15:16]
  mean = sums[:, 0:3] / jnp.maximum(counts, 1.0)
  zc = mean - jnp.max(mean, axis=1, keepdims=True)
  out_ref[...] = zc - jnp.log(jnp.sum(jnp.exp(zc), axis=1, keepdims=True))


def kernel(x, edge_index, batch, W1, b1, W2, b2, W3, b3, W4, b4):
  f32 = jnp.float32
  row = edge_index[0].astype(jnp.int32)
  col = edge_index[1].astype(jnp.int32)
  pad_idx = N + (jnp.arange(EP - E, dtype=jnp.int32) % PAD_SPREAD)
  rows_p = jnp.concatenate([row, pad_idx]).reshape(NW, CPW, CHUNK)
  cols_p = jnp.concatenate([col, pad_idx]).reshape(NW, CPW, CHUNK)

  x_pad = jnp.pad(x.astype(f32), ((0, N_PAD - N), (0, 16 - x.shape[1])))
  batch_pad = jnp.pad(batch.astype(jnp.int32), (0, N_PAD - N),
                      constant_values=G).reshape(N_PAD, 1)
  w1p = jnp.pad(W1, ((0, 16 - W1.shape[0]), (0, 0)))
  b1r = b1.reshape(1, 128)
  b2r = b2.reshape(1, 64)
  w2s = jnp.stack([W2[:, :32], W2[:, 32:]])
  w3p = jnp.pad(W3, ((0, 0), (0, 32 - W3.shape[1])))
  b3p = jnp.pad(b3, (0, 32 - b3.shape[0])).reshape(1, 32)
  w4p = jnp.pad(W4, ((0, 32 - W4.shape[0]), (0, 16 - W4.shape[1])))
  b4p = jnp.pad(b4, (0, 16 - b4.shape[0])).reshape(1, 16)

  deg_p = _sc_deg(cols_p)
  deg_t = deg_p.T

  dinv, xp = _tc_call(
      _tc1_body,
      (jax.ShapeDtypeStruct((N_PAD, 1), f32),
       jax.ShapeDtypeStruct((1, N_PAD, 16), f32)))(deg_t, x_pad)

  p1 = _agg16(xp, rows_p, cols_p)
  z2 = _tc_call(
      _tc2_body,
      jax.ShapeDtypeStruct((2, N_PAD, 32), f32))(p1, xp, dinv, w1p, b1r, w2s)

  p2 = _agg32x2(z2, rows_p, cols_p)
  z3 = _tc_call(
      _tc3_body,
      jax.ShapeDtypeStruct((1, N_PAD, 32), f32))(p2, z2, dinv, b2r, w3p)

  p3 = _agg32(z3, rows_p, cols_p)
  z4 = _tc_call(
      _tc4_body,
      jax.ShapeDtypeStruct((1, N_PAD, 16), f32))(p3, z3, dinv, b3p, w4p)

  p4 = _agg16(z4, rows_p, cols_p)
  out = _tc_call(
      _tc5_body,
      jax.ShapeDtypeStruct((G, 3), f32))(p4, z4, dinv, b4p, batch_pad)
  return out
